# baseline (speedup 1.0000x reference)
import functools

import jax
import jax.numpy as jnp
from jax import lax
from jax.experimental import pallas as pl
from jax.experimental.pallas import tpu as pltpu

_THRESHOLD = 0.5
_RB = 32
_HALO = 6


def _mm(sl3, w):
    m = sl3.shape[0] * sl3.shape[1]
    return lax.dot_general(
        sl3.reshape(m, sl3.shape[2]), w,
        (((1,), (0,)), ((), ())),
        preferred_element_type=jnp.float32,
    )


def _body(xp_hbm, cm_ref, w1_ref, w2_ref, wd_ref, wp_ref, b_ref,
          o_ref, xs_ref, f1_ref, sem, *, rb, H, W):
    b = pl.program_id(0)
    i = pl.program_id(1)
    r0 = i * rb
    C = xs_ref.shape[-1]

    cp = pltpu.make_async_copy(
        xp_hbm.at[b, pl.ds(r0, rb + 2 * _HALO)], xs_ref, sem)
    cp.start()
    cp.wait()

    a = rb + 8
    acc = None
    for ky in range(3):
        for kx in range(3):
            oy, ox = (ky - 1) * 2, (kx - 1) * 2
            sl = xs_ref[2 + oy:2 + oy + a, 6 + ox:6 + ox + W, :]
            c = _mm(sl, w1_ref[ky * 3 + kx])
            acc = c if acc is None else acc + c
    xc = xs_ref[2:2 + a, 6:6 + W, :].reshape(a * W, C)
    fine1 = jax.nn.relu(xc + acc + b_ref[0])
    rows = lax.broadcasted_iota(jnp.int32, (a * W, 1), 0) // W + (r0 - 4)
    valid = jnp.logical_and(rows >= 0, rows < H)
    fine1 = jnp.where(valid, fine1, 0.0)
    f1_ref[:, 0:6, :] = jnp.zeros((a, 6, C), jnp.float32)
    f1_ref[:, 6 + W:12 + W, :] = jnp.zeros((a, 6, C), jnp.float32)
    f1_ref[:, 6:6 + W, :] = fine1.reshape(a, W, C)

    acc2 = None
    for ky in range(3):
        for kx in range(3):
            oy, ox = (ky - 1) * 4, (kx - 1) * 4
            sl = f1_ref[4 + oy:4 + oy + rb, 6 + ox:6 + ox + W, :]
            c = _mm(sl, w2_ref[ky * 3 + kx])
            acc2 = c if acc2 is None else acc2 + c
    f1c = f1_ref[4:4 + rb, 6:6 + W, :].reshape(rb * W, C)
    fine2 = jax.nn.relu(f1c + acc2 + b_ref[1])

    dw = None
    for ky in range(3):
        for kx in range(3):
            oy, ox = ky - 1, kx - 1
            sl = xs_ref[6 + oy:6 + oy + rb, 6 + ox:6 + ox + W, :]
            c = sl.reshape(rb * W, C) * wd_ref[ky * 3 + kx]
            dw = c if dw is None else dw + c
    coarse1 = jax.nn.relu(dw + b_ref[2])
    coarse2 = jax.nn.relu(
        lax.dot_general(coarse1, wp_ref[...], (((1,), (0,)), ((), ())),
                        preferred_element_type=jnp.float32) + b_ref[3])

    sel = cm_ref[0] > _THRESHOLD
    o_ref[0] = jnp.where(sel, fine2, coarse2).reshape(rb, W, C)


def kernel(x, complexity_map, w1, b1, w2, b2, wd, bd, wp, bp):
    B, C, H, W = x.shape
    rb = min(_RB, H)

    xt = jnp.transpose(x, (0, 2, 3, 1))
    xp = jnp.pad(xt, ((0, 0), (_HALO, _HALO), (_HALO, _HALO), (0, 0)))
    w1t = jnp.transpose(w1, (2, 3, 1, 0)).reshape(9, C, C)
    w2t = jnp.transpose(w2, (2, 3, 1, 0)).reshape(9, C, C)
    wdt = jnp.transpose(wd[:, 0], (1, 2, 0)).reshape(9, C)
    wpt = wp[:, :, 0, 0].T
    bias = jnp.stack([b1, b2, bd, bp])

    body = functools.partial(_body, rb=rb, H=H, W=W)
    out = pl.pallas_call(
        body,
        grid=(B, H // rb),
        in_specs=[
            pl.BlockSpec(memory_space=pl.ANY),
            pl.BlockSpec((1, rb * W, 1), lambda b, i: (b, i, 0)),
            pl.BlockSpec((9, C, C), lambda b, i: (0, 0, 0)),
            pl.BlockSpec((9, C, C), lambda b, i: (0, 0, 0)),
            pl.BlockSpec((9, C), lambda b, i: (0, 0)),
            pl.BlockSpec((C, C), lambda b, i: (0, 0)),
            pl.BlockSpec((4, C), lambda b, i: (0, 0)),
        ],
        out_specs=pl.BlockSpec((1, rb, W, C), lambda b, i: (b, i, 0, 0)),
        out_shape=jax.ShapeDtypeStruct((B, H, W, C), jnp.float32),
        scratch_shapes=[
            pltpu.VMEM((rb + 2 * _HALO, W + 2 * _HALO, C), jnp.float32),
            pltpu.VMEM((rb + 8, W + 12, C), jnp.float32),
            pltpu.SemaphoreType.DMA,
        ],
        compiler_params=pltpu.CompilerParams(
            dimension_semantics=("parallel", "arbitrary"),
            vmem_limit_bytes=56 * 1024 * 1024,
        ),
        name="arfr_fused",
    )(xp, complexity_map.reshape(B, H * W, 1), w1t, w2t, wdt, wpt, bias)
    return jnp.transpose(out, (0, 3, 1, 2))


# speedup vs baseline: 1.0516x; 1.0516x over previous
import functools

import jax
import jax.numpy as jnp
from jax import lax
from jax.experimental import pallas as pl
from jax.experimental.pallas import tpu as pltpu

_THRESHOLD = 0.5
_RB = 32
_LP = 10
_FP = 12


def _mm(sl3, w):
    m = sl3.shape[0] * sl3.shape[1]
    return lax.dot_general(
        sl3.reshape(m, sl3.shape[2]), w,
        (((1,), (0,)), ((), ())),
        preferred_element_type=jnp.float32,
    )


def _body(xp_hbm, cm_ref, w1_ref, w2_ref, wd_ref, wp_ref, b_ref,
          o_ref, xs_ref, f1_ref, sem, *, rb, H, W):
    b = pl.program_id(0)
    i = pl.program_id(1)
    r0 = i * rb
    C = xs_ref.shape[-1]
    Ws = W + 8

    cp = pltpu.make_async_copy(
        xp_hbm.at[b, pl.ds(r0, rb + 12)], xs_ref, sem)
    cp.start()
    cp.wait()

    a = rb + 8
    x_cat = jnp.concatenate(
        [xs_ref[2 + (ky - 1) * 2:2 + (ky - 1) * 2 + a, 8:8 + Ws, :]
         .reshape(a * Ws, C) for ky in range(3)], axis=1)
    acc = None
    for kx in range(3):
        g = lax.dot_general(x_cat, w1_ref[kx], (((1,), (0,)), ((), ())),
                            preferred_element_type=jnp.float32)
        ox = (kx - 1) * 2
        gs = g.reshape(a, Ws, C)[:, 2 + ox:2 + ox + W, :].reshape(a * W, C)
        acc = gs if acc is None else acc + gs
    xc = xs_ref[2:2 + a, _LP:_LP + W, :].reshape(a * W, C)
    fine1 = jax.nn.relu(xc + acc + b_ref[0])
    rows = lax.broadcasted_iota(jnp.int32, (a * W, 1), 0) // W + (r0 - 4)
    valid = jnp.logical_and(rows >= 0, rows < H)
    fine1 = jnp.where(valid, fine1, 0.0)
    f1_ref[:, 8:_FP, :] = jnp.zeros((a, _FP - 8, C), jnp.float32)
    f1_ref[:, _FP + W:_FP + W + 4, :] = jnp.zeros((a, 4, C), jnp.float32)
    f1_ref[:, _FP:_FP + W, :] = fine1.reshape(a, W, C)

    f_cat = jnp.concatenate(
        [f1_ref[4 + (ky - 1) * 4:4 + (ky - 1) * 4 + rb, 8:8 + Ws, :]
         .reshape(rb * Ws, C) for ky in range(3)], axis=1)
    acc2 = None
    for kx in range(3):
        g = lax.dot_general(f_cat, w2_ref[kx], (((1,), (0,)), ((), ())),
                            preferred_element_type=jnp.float32)
        ox = (kx - 1) * 4
        gs = g.reshape(rb, Ws, C)[:, 4 + ox:4 + ox + W, :].reshape(rb * W, C)
        acc2 = gs if acc2 is None else acc2 + gs
    f1c = f1_ref[4:4 + rb, _FP:_FP + W, :].reshape(rb * W, C)
    fine2 = jax.nn.relu(f1c + acc2 + b_ref[1])

    s_oy = [xs_ref[5 + ky:5 + ky + rb, 8:8 + Ws, :].reshape(rb * Ws, C)
            for ky in range(3)]
    dw = None
    for kx in range(3):
        t = None
        for ky in range(3):
            c = s_oy[ky] * wd_ref[ky * 3 + kx]
            t = c if t is None else t + c
        ox = kx - 1
        ts = t.reshape(rb, Ws, C)[:, 2 + ox:2 + ox + W, :].reshape(rb * W, C)
        dw = ts if dw is None else dw + ts
    coarse1 = jax.nn.relu(dw + b_ref[2])
    coarse2 = jax.nn.relu(
        lax.dot_general(coarse1, wp_ref[...], (((1,), (0,)), ((), ())),
                        preferred_element_type=jnp.float32) + b_ref[3])

    sel = cm_ref[0] > _THRESHOLD
    o_ref[0] = jnp.where(sel, fine2, coarse2).reshape(rb, W, C)


def kernel(x, complexity_map, w1, b1, w2, b2, wd, bd, wp, bp):
    B, C, H, W = x.shape
    rb = min(_RB, H)

    xt = jnp.transpose(x, (0, 2, 3, 1))
    xp = jnp.pad(xt, ((0, 0), (6, 6), (_LP, 6), (0, 0)))
    w1t = jnp.transpose(w1, (3, 2, 1, 0)).reshape(3, 3 * C, C)
    w2t = jnp.transpose(w2, (3, 2, 1, 0)).reshape(3, 3 * C, C)
    wdt = jnp.transpose(wd[:, 0], (1, 2, 0)).reshape(9, C)
    wpt = wp[:, :, 0, 0].T
    bias = jnp.stack([b1, b2, bd, bp])

    body = functools.partial(_body, rb=rb, H=H, W=W)
    out = pl.pallas_call(
        body,
        grid=(B, H // rb),
        in_specs=[
            pl.BlockSpec(memory_space=pl.ANY),
            pl.BlockSpec((1, rb * W, 1), lambda b, i: (b, i, 0)),
            pl.BlockSpec((3, 3 * C, C), lambda b, i: (0, 0, 0)),
            pl.BlockSpec((3, 3 * C, C), lambda b, i: (0, 0, 0)),
            pl.BlockSpec((9, C), lambda b, i: (0, 0)),
            pl.BlockSpec((C, C), lambda b, i: (0, 0)),
            pl.BlockSpec((4, C), lambda b, i: (0, 0)),
        ],
        out_specs=pl.BlockSpec((1, rb, W, C), lambda b, i: (b, i, 0, 0)),
        out_shape=jax.ShapeDtypeStruct((B, H, W, C), jnp.float32),
        scratch_shapes=[
            pltpu.VMEM((rb + 12, W + _LP + 6, C), jnp.float32),
            pltpu.VMEM((rb + 8, W + _FP + 4, C), jnp.float32),
            pltpu.SemaphoreType.DMA,
        ],
        compiler_params=pltpu.CompilerParams(
            dimension_semantics=("parallel", "arbitrary"),
            vmem_limit_bytes=56 * 1024 * 1024,
        ),
        name="arfr_fused",
    )(xp, complexity_map.reshape(B, H * W, 1), w1t, w2t, wdt, wpt, bias)
    return jnp.transpose(out, (0, 3, 1, 2))


# speedup vs baseline: 1.1803x; 1.1224x over previous
import functools

import jax
import jax.numpy as jnp
from jax import lax
from jax.experimental import pallas as pl
from jax.experimental.pallas import tpu as pltpu

_THRESHOLD = 0.5
_RB = 32
_LP = 10
_FP = 12


def _body(xf_hbm, cm_ref, w1_ref, w2_ref, wd_ref, wp_ref, b_ref,
          o_ref, xr_ref, xs_ref, f1_ref, sem, *, rb, H, W):
    b = pl.program_id(0)
    i = pl.program_id(1)
    r0 = i * rb
    C = xr_ref.shape[0]
    Ws = W + 8
    nr = H // rb

    def _cp(src, dst):
        cp = pltpu.make_async_copy(src, dst, sem)
        cp.start()
        cp.wait()

    if nr == 1:
        xr_ref[:, :6 * W] = jnp.zeros((C, 6 * W), jnp.float32)
        xr_ref[:, (rb + 6) * W:] = jnp.zeros((C, 6 * W), jnp.float32)
        _cp(xf_hbm.at[b, :, pl.ds(0, rb * W)],
            xr_ref.at[:, pl.ds(6 * W, rb * W)])
    else:
        @pl.when(i == 0)
        def _():
            xr_ref[:, :6 * W] = jnp.zeros((C, 6 * W), jnp.float32)
            _cp(xf_hbm.at[b, :, pl.ds(0, (rb + 6) * W)],
                xr_ref.at[:, pl.ds(6 * W, (rb + 6) * W)])

        @pl.when(i == nr - 1)
        def _():
            xr_ref[:, (rb + 6) * W:] = jnp.zeros((C, 6 * W), jnp.float32)
            _cp(xf_hbm.at[b, :, pl.ds((H - rb - 6) * W, (rb + 6) * W)],
                xr_ref.at[:, pl.ds(0, (rb + 6) * W)])

        if nr > 2:
            @pl.when(jnp.logical_and(i > 0, i < nr - 1))
            def _():
                _cp(xf_hbm.at[b, :, pl.ds((r0 - 6) * W, (rb + 12) * W)],
                    xr_ref.at[:, :])

    x3 = lax.transpose(xr_ref[...], (1, 0)).reshape(rb + 12, W, C)
    xs_ref[:, 8:_LP, :] = jnp.zeros((rb + 12, _LP - 8, C), jnp.float32)
    xs_ref[:, _LP + W:, :] = jnp.zeros((rb + 12, 6, C), jnp.float32)
    xs_ref[:, _LP:_LP + W, :] = x3

    a = rb + 8
    x_cat = jnp.concatenate(
        [xs_ref[2 + (ky - 1) * 2:2 + (ky - 1) * 2 + a, 8:8 + Ws, :]
         .reshape(a * Ws, C) for ky in range(3)], axis=1)
    acc = None
    for kx in range(3):
        g = lax.dot_general(x_cat, w1_ref[kx], (((1,), (0,)), ((), ())),
                            preferred_element_type=jnp.float32)
        ox = (kx - 1) * 2
        gs = g.reshape(a, Ws, C)[:, 2 + ox:2 + ox + W, :].reshape(a * W, C)
        acc = gs if acc is None else acc + gs
    xc = xs_ref[2:2 + a, _LP:_LP + W, :].reshape(a * W, C)
    fine1 = jax.nn.relu(xc + acc + b_ref[0])
    rows = lax.broadcasted_iota(jnp.int32, (a * W, 1), 0) // W + (r0 - 4)
    valid = jnp.logical_and(rows >= 0, rows < H)
    fine1 = jnp.where(valid, fine1, 0.0)
    f1_ref[:, 8:_FP, :] = jnp.zeros((a, _FP - 8, C), jnp.float32)
    f1_ref[:, _FP + W:_FP + W + 4, :] = jnp.zeros((a, 4, C), jnp.float32)
    f1_ref[:, _FP:_FP + W, :] = fine1.reshape(a, W, C)

    f_cat = jnp.concatenate(
        [f1_ref[4 + (ky - 1) * 4:4 + (ky - 1) * 4 + rb, 8:8 + Ws, :]
         .reshape(rb * Ws, C) for ky in range(3)], axis=1)
    acc2 = None
    for kx in range(3):
        g = lax.dot_general(f_cat, w2_ref[kx], (((1,), (0,)), ((), ())),
                            preferred_element_type=jnp.float32)
        ox = (kx - 1) * 4
        gs = g.reshape(rb, Ws, C)[:, 4 + ox:4 + ox + W, :].reshape(rb * W, C)
        acc2 = gs if acc2 is None else acc2 + gs
    f1c = f1_ref[4:4 + rb, _FP:_FP + W, :].reshape(rb * W, C)
    fine2 = jax.nn.relu(f1c + acc2 + b_ref[1])

    s_oy = [xs_ref[5 + ky:5 + ky + rb, 8:8 + Ws, :].reshape(rb * Ws, C)
            for ky in range(3)]
    dw = None
    for kx in range(3):
        t = None
        for ky in range(3):
            c = s_oy[ky] * wd_ref[ky * 3 + kx]
            t = c if t is None else t + c
        ox = kx - 1
        ts = t.reshape(rb, Ws, C)[:, 2 + ox:2 + ox + W, :].reshape(rb * W, C)
        dw = ts if dw is None else dw + ts
    coarse1 = jax.nn.relu(dw + b_ref[2])
    coarse2 = jax.nn.relu(
        lax.dot_general(coarse1, wp_ref[...], (((1,), (0,)), ((), ())),
                        preferred_element_type=jnp.float32) + b_ref[3])

    sel = cm_ref[0] > _THRESHOLD
    res = jnp.where(sel, fine2, coarse2)
    o_ref[0] = lax.transpose(res, (1, 0))


def kernel(x, complexity_map, w1, b1, w2, b2, wd, bd, wp, bp):
    B, C, H, W = x.shape
    rb = min(_RB, H)

    w1t = jnp.transpose(w1, (3, 2, 1, 0)).reshape(3, 3 * C, C)
    w2t = jnp.transpose(w2, (3, 2, 1, 0)).reshape(3, 3 * C, C)
    wdt = jnp.transpose(wd[:, 0], (1, 2, 0)).reshape(9, C)
    wpt = wp[:, :, 0, 0].T
    bias = jnp.stack([b1, b2, bd, bp])

    body = functools.partial(_body, rb=rb, H=H, W=W)
    out = pl.pallas_call(
        body,
        grid=(B, H // rb),
        in_specs=[
            pl.BlockSpec(memory_space=pl.ANY),
            pl.BlockSpec((1, rb * W, 1), lambda b, i: (b, i, 0)),
            pl.BlockSpec((3, 3 * C, C), lambda b, i: (0, 0, 0)),
            pl.BlockSpec((3, 3 * C, C), lambda b, i: (0, 0, 0)),
            pl.BlockSpec((9, C), lambda b, i: (0, 0)),
            pl.BlockSpec((C, C), lambda b, i: (0, 0)),
            pl.BlockSpec((4, C), lambda b, i: (0, 0)),
        ],
        out_specs=pl.BlockSpec((1, C, rb * W), lambda b, i: (b, 0, i)),
        out_shape=jax.ShapeDtypeStruct((B, C, H * W), jnp.float32),
        scratch_shapes=[
            pltpu.VMEM((C, (rb + 12) * W), jnp.float32),
            pltpu.VMEM((rb + 12, W + _LP + 6, C), jnp.float32),
            pltpu.VMEM((rb + 8, W + _FP + 4, C), jnp.float32),
            pltpu.SemaphoreType.DMA,
        ],
        compiler_params=pltpu.CompilerParams(
            dimension_semantics=("parallel", "arbitrary"),
            vmem_limit_bytes=56 * 1024 * 1024,
        ),
        name="arfr_fused",
    )(x.reshape(B, C, H * W), complexity_map.reshape(B, H * W, 1),
      w1t, w2t, wdt, wpt, bias)
    return out.reshape(B, C, H, W)
